# Initial kernel scaffold; baseline (speedup 1.0000x reference)
import jax
import jax.numpy as jnp
from jax.experimental import pallas as pl


def kernel(x, edge_index, edge_attr, params):
    raise NotImplementedError("write your pallas kernel here")


# speedup vs baseline: 2.0775x; 2.0775x over previous
import functools

import jax
import jax.numpy as jnp
from jax import lax
from jax.experimental import pallas as pl
from jax.experimental.pallas import tpu as pltpu
from jax.experimental.pallas import tpu_sc as plsc

N = 10000
E = 320000
H = 128
NC = 2
NS = 16
NW = NC * NS
PER_W = E // NW
C = 80
CH = PER_W // C
NPAD = 10240
ROWS_PER_TILE = NPAD // NS

BN_EPS = 1e-5

def _mk_mesh():
    return plsc.VectorSubcoreMesh(core_axis_name="c", subcore_axis_name="s",
                                  num_cores=NC, num_subcores=NS)


def _sc_gather1(table, idx):
    @functools.partial(
        pl.kernel,
        out_type=jax.ShapeDtypeStruct((E, H), jnp.float32),
        mesh=_mk_mesh(),
        scratch_types=[
            pltpu.VMEM((C,), jnp.int32),
            pltpu.VMEM((C, H), jnp.float32),
            pltpu.SemaphoreType.DMA,
        ],
    )
    def k(table_hbm, idx_hbm, out_hbm, idxv, rows, sem):
        wid = lax.axis_index("s") * NC + lax.axis_index("c")
        base = wid * PER_W

        def step(i, carry):
            off = base + i * C
            pltpu.sync_copy(idx_hbm.at[pl.ds(off, C)], idxv)
            pltpu.async_copy(table_hbm.at[idxv], rows, sem).wait()
            pltpu.sync_copy(rows, out_hbm.at[pl.ds(off, C)])
            return carry

        lax.fori_loop(0, CH, step, 0)

    return k(table, idx)


def _sc_gather3(xn, t2, row, col):
    @functools.partial(
        pl.kernel,
        out_type=(
            jax.ShapeDtypeStruct((E, H), jnp.float32),
            jax.ShapeDtypeStruct((E, H), jnp.float32),
            jax.ShapeDtypeStruct((E, H), jnp.float32),
        ),
        mesh=_mk_mesh(),
        scratch_types=[
            pltpu.VMEM((C,), jnp.int32),
            pltpu.VMEM((C,), jnp.int32),
            pltpu.VMEM((C, H), jnp.float32),
            pltpu.VMEM((C, H), jnp.float32),
            pltpu.VMEM((C, H), jnp.float32),
            pltpu.SemaphoreType.DMA,
        ],
    )
    def k(xn_hbm, t2_hbm, row_hbm, col_hbm, src_hbm, tgt_hbm, g2_hbm,
          rowv, colv, b1, b2, b3, sem):
        wid = lax.axis_index("s") * NC + lax.axis_index("c")
        base = wid * PER_W

        def step(i, carry):
            off = base + i * C
            pltpu.sync_copy(row_hbm.at[pl.ds(off, C)], rowv)
            pltpu.sync_copy(col_hbm.at[pl.ds(off, C)], colv)
            pltpu.async_copy(xn_hbm.at[rowv], b1, sem).wait()
            pltpu.async_copy(xn_hbm.at[colv], b2, sem).wait()
            pltpu.async_copy(t2_hbm.at[rowv], b3, sem).wait()
            pltpu.sync_copy(b1, src_hbm.at[pl.ds(off, C)])
            pltpu.sync_copy(b2, tgt_hbm.at[pl.ds(off, C)])
            pltpu.sync_copy(b3, g2_hbm.at[pl.ds(off, C)])
            return carry

        lax.fori_loop(0, CH, step, 0)

    return k(xn, t2, row, col)


def _sc_scatter_add(msg, col, zeros_tile):
    @functools.partial(
        pl.kernel,
        out_type=jax.ShapeDtypeStruct((NC, NPAD, H), jnp.float32),
        mesh=_mk_mesh(),
        scratch_types=[
            pltpu.VMEM((C,), jnp.int32),
            pltpu.VMEM((C, H), jnp.float32),
            pltpu.VMEM_SHARED((NPAD, H), jnp.float32),
        ],
    )
    def k(msg_hbm, col_hbm, zero_hbm, p_hbm, colv, rows, acc):
        cid = lax.axis_index("c")
        sid = lax.axis_index("s")
        wid = sid * NC + cid
        base = wid * PER_W
        pltpu.sync_copy(zero_hbm, acc.at[pl.ds(sid * ROWS_PER_TILE, ROWS_PER_TILE)])
        plsc.subcore_barrier()

        def step(i, carry):
            off = base + i * C
            pltpu.sync_copy(col_hbm.at[pl.ds(off, C)], colv)
            pltpu.sync_copy(msg_hbm.at[pl.ds(off, C)], rows)
            pltpu.sync_copy(rows, acc.at[colv], add=True)
            return carry

        lax.fori_loop(0, CH, step, 0)
        plsc.subcore_barrier()
        pltpu.sync_copy(
            acc.at[pl.ds(sid * ROWS_PER_TILE, ROWS_PER_TILE)],
            p_hbm.at[cid].at[pl.ds(sid * ROWS_PER_TILE, ROWS_PER_TILE)])

    return k(msg, col, zeros_tile)


BN_TILE = 1000
BE = 2000


def _relu(v):
    return jnp.maximum(v, 0.0)


def _dot(a, b):
    return jnp.dot(a, b, preferred_element_type=jnp.float32)


def _tc_node_encode(x, npw, npb, pnw, pnb):
    def k(x_ref, npw_ref, npb_ref, pnw_ref, pnb_ref, x0_ref, t1_ref):
        x0 = _relu(_dot(x_ref[...], npw_ref[...]) + npb_ref[...])
        x0_ref[...] = x0
        t1_ref[...] = _relu(_dot(x0, pnw_ref[...]) + pnb_ref[...])

    g = N // BN_TILE
    return pl.pallas_call(
        k,
        grid=(g,),
        in_specs=[
            pl.BlockSpec((BN_TILE, H), lambda i: (i, 0)),
            pl.BlockSpec((H, H), lambda i: (0, 0)),
            pl.BlockSpec((1, H), lambda i: (0, 0)),
            pl.BlockSpec((H, H), lambda i: (0, 0)),
            pl.BlockSpec((1, H), lambda i: (0, 0)),
        ],
        out_specs=[
            pl.BlockSpec((BN_TILE, H), lambda i: (i, 0)),
            pl.BlockSpec((BN_TILE, H), lambda i: (i, 0)),
        ],
        out_shape=[
            jax.ShapeDtypeStruct((N, H), jnp.float32),
            jax.ShapeDtypeStruct((N, H), jnp.float32),
        ],
    )(x, npw, npb, pnw, pnb)


def _tc_edge_encode(edge_attr, epw, epb):
    def k(ea_ref, w_ref, b_ref, out_ref):
        out_ref[...] = _relu(_dot(ea_ref[...], w_ref[...]) + b_ref[...])

    g = E // BE
    d_edge = edge_attr.shape[1]
    return pl.pallas_call(
        k,
        grid=(g,),
        in_specs=[
            pl.BlockSpec((BE, d_edge), lambda i: (i, 0)),
            pl.BlockSpec((d_edge, H), lambda i: (0, 0)),
            pl.BlockSpec((1, H), lambda i: (0, 0)),
        ],
        out_specs=pl.BlockSpec((BE, H), lambda i: (i, 0)),
        out_shape=jax.ShapeDtypeStruct((E, H), jnp.float32),
    )(edge_attr, epw, epb)


def _tc_message(ea0, g1, pw1, pb1, pw2, pb2):
    def k(ea_ref, g_ref, w1_ref, b1_ref, w2_ref, b2_ref, m_ref):
        h = _relu(_dot(ea_ref[...], w1_ref[...]) + b1_ref[...])
        om = jnp.sum(h * w2_ref[...], axis=-1, keepdims=True) + b2_ref[...]
        m_ref[...] = om * g_ref[...]

    g = E // BE
    return pl.pallas_call(
        k,
        grid=(g,),
        in_specs=[
            pl.BlockSpec((BE, H), lambda i: (i, 0)),
            pl.BlockSpec((BE, H), lambda i: (i, 0)),
            pl.BlockSpec((H, 32), lambda i: (0, 0)),
            pl.BlockSpec((1, 32), lambda i: (0, 0)),
            pl.BlockSpec((1, 32), lambda i: (0, 0)),
            pl.BlockSpec((1, 1), lambda i: (0, 0)),
        ],
        out_specs=pl.BlockSpec((BE, H), lambda i: (i, 0)),
        out_shape=jax.ShapeDtypeStruct((E, H), jnp.float32),
    )(ea0, g1, pw1, pb1, pw2, pb2)


def _tc_sum_stats(xin, partials):
    def k(x_ref, p_ref, s_ref, st_ref):
        s = x_ref[...] + p_ref[0] + p_ref[1]
        s_ref[...] = s
        upd = jnp.concatenate(
            [jnp.sum(s, 0)[None], jnp.sum(s * s, 0)[None],
             jnp.zeros((6, H), jnp.float32)], axis=0)

        @pl.when(pl.program_id(0) == 0)
        def _():
            st_ref[...] = jnp.zeros_like(st_ref)

        st_ref[...] += upd

    g = N // BN_TILE
    return pl.pallas_call(
        k,
        grid=(g,),
        in_specs=[
            pl.BlockSpec((BN_TILE, H), lambda i: (i, 0)),
            pl.BlockSpec((NC, BN_TILE, H), lambda i: (0, i, 0)),
        ],
        out_specs=[
            pl.BlockSpec((BN_TILE, H), lambda i: (i, 0)),
            pl.BlockSpec((8, H), lambda i: (0, 0)),
        ],
        out_shape=[
            jax.ShapeDtypeStruct((N, H), jnp.float32),
            jax.ShapeDtypeStruct((8, H), jnp.float32),
        ],
    )(xin, partials)


def _tc_bn_relu_node(s, stats, gamma, beta, pnw=None, pnb=None):
    with_t = pnw is not None

    def k(*refs):
        if with_t:
            s_ref, st_ref, g_ref, b_ref, w_ref, wb_ref, xn_ref, t_ref = refs
        else:
            s_ref, st_ref, g_ref, b_ref, xn_ref = refs
        st = st_ref[...]
        mu = st[0:1] * (1.0 / N)
        var = st[1:2] * (1.0 / N) - mu * mu
        xn = _relu(g_ref[...] * (s_ref[...] - mu) * lax.rsqrt(var + BN_EPS)
                   + b_ref[...])
        xn_ref[...] = xn
        if with_t:
            t_ref[...] = _relu(_dot(xn, w_ref[...]) + wb_ref[...])

    g = N // BN_TILE
    in_specs = [
        pl.BlockSpec((BN_TILE, H), lambda i: (i, 0)),
        pl.BlockSpec((8, H), lambda i: (0, 0)),
        pl.BlockSpec((1, H), lambda i: (0, 0)),
        pl.BlockSpec((1, H), lambda i: (0, 0)),
    ]
    args = [s, stats, gamma, beta]
    out_specs = [pl.BlockSpec((BN_TILE, H), lambda i: (i, 0))]
    out_shape = [jax.ShapeDtypeStruct((N, H), jnp.float32)]
    if with_t:
        in_specs += [pl.BlockSpec((H, H), lambda i: (0, 0)),
                     pl.BlockSpec((1, H), lambda i: (0, 0))]
        args += [pnw, pnb]
        out_specs.append(pl.BlockSpec((BN_TILE, H), lambda i: (i, 0)))
        out_shape.append(jax.ShapeDtypeStruct((N, H), jnp.float32))
    res = pl.pallas_call(
        k, grid=(g,), in_specs=in_specs, out_specs=out_specs,
        out_shape=out_shape,
    )(*args)
    return res if with_t else res[0]


def _tc_edge_mlp(ea0, src, tgt, w_ea, w_src, w_tgt, w_ds, w_cos, b1, w2, b2):
    def k(ea_ref, s_ref, t_ref, wea_ref, wsrc_ref, wtgt_ref, wds_ref,
          wcos_ref, b1_ref, w2_ref, b2_ref, y_ref, st_ref):
        s = s_ref[...]
        t = t_ref[...]
        d = jnp.abs(s - t)
        sn = jnp.sqrt(jnp.sum(s * s, axis=-1, keepdims=True))
        tn = jnp.sqrt(jnp.sum(t * t, axis=-1, keepdims=True))
        dot = jnp.sum(s * t, axis=-1, keepdims=True)
        cos = dot / jnp.maximum(sn * tn, 1e-8)
        h = _relu(_dot(ea_ref[...], wea_ref[...]) + _dot(s, wsrc_ref[...])
                  + _dot(t, wtgt_ref[...]) + _dot(d, wds_ref[...])
                  + cos * wcos_ref[...] + b1_ref[...])
        y = _dot(h, w2_ref[...]) + b2_ref[...]
        y_ref[...] = y
        upd = jnp.concatenate(
            [jnp.sum(y, 0)[None], jnp.sum(y * y, 0)[None],
             jnp.zeros((6, H), jnp.float32)], axis=0)

        @pl.when(pl.program_id(0) == 0)
        def _():
            st_ref[...] = jnp.zeros_like(st_ref)

        st_ref[...] += upd

    g = E // BE
    return pl.pallas_call(
        k,
        grid=(g,),
        in_specs=[
            pl.BlockSpec((BE, H), lambda i: (i, 0)),
            pl.BlockSpec((BE, H), lambda i: (i, 0)),
            pl.BlockSpec((BE, H), lambda i: (i, 0)),
            pl.BlockSpec((H, H), lambda i: (0, 0)),
            pl.BlockSpec((H, H), lambda i: (0, 0)),
            pl.BlockSpec((H, H), lambda i: (0, 0)),
            pl.BlockSpec((H, H), lambda i: (0, 0)),
            pl.BlockSpec((1, H), lambda i: (0, 0)),
            pl.BlockSpec((1, H), lambda i: (0, 0)),
            pl.BlockSpec((H, H), lambda i: (0, 0)),
            pl.BlockSpec((1, H), lambda i: (0, 0)),
        ],
        out_specs=[
            pl.BlockSpec((BE, H), lambda i: (i, 0)),
            pl.BlockSpec((8, H), lambda i: (0, 0)),
        ],
        out_shape=[
            jax.ShapeDtypeStruct((E, H), jnp.float32),
            jax.ShapeDtypeStruct((8, H), jnp.float32),
        ],
    )(ea0, src, tgt, w_ea, w_src, w_tgt, w_ds, w_cos, b1, w2, b2)


def _tc_message2(y, stats, gamma, beta, pw1, pb1, pw2, pb2, g2):
    def k(y_ref, st_ref, g_ref, b_ref, w1_ref, b1_ref, w2_ref, b2_ref,
          g2_ref, m_ref):
        st = st_ref[...]
        mu = st[0:1] * (1.0 / E)
        var = st[1:2] * (1.0 / E) - mu * mu
        ea1 = _relu(g_ref[...] * (y_ref[...] - mu) * lax.rsqrt(var + BN_EPS)
                    + b_ref[...])
        h = _relu(_dot(ea1, w1_ref[...]) + b1_ref[...])
        om = jnp.sum(h * w2_ref[...], axis=-1, keepdims=True) + b2_ref[...]
        m_ref[...] = om * g2_ref[...]

    g = E // BE
    return pl.pallas_call(
        k,
        grid=(g,),
        in_specs=[
            pl.BlockSpec((BE, H), lambda i: (i, 0)),
            pl.BlockSpec((8, H), lambda i: (0, 0)),
            pl.BlockSpec((1, H), lambda i: (0, 0)),
            pl.BlockSpec((1, H), lambda i: (0, 0)),
            pl.BlockSpec((H, 32), lambda i: (0, 0)),
            pl.BlockSpec((1, 32), lambda i: (0, 0)),
            pl.BlockSpec((1, 32), lambda i: (0, 0)),
            pl.BlockSpec((1, 1), lambda i: (0, 0)),
            pl.BlockSpec((BE, H), lambda i: (i, 0)),
        ],
        out_specs=pl.BlockSpec((BE, H), lambda i: (i, 0)),
        out_shape=jax.ShapeDtypeStruct((E, H), jnp.float32),
    )(y, stats, gamma, beta, pw1, pb1, pw2, pb2, g2)


def kernel(x, edge_index, edge_attr, params):
    row = edge_index[0]
    col = edge_index[1]
    p = params
    b0, b1 = p['blocks'][0], p['blocks'][1]

    def r2(v):
        return v.reshape(1, -1)

    zeros_tile = jnp.zeros((ROWS_PER_TILE, H), jnp.float32)

    x0, t1 = _tc_node_encode(x, p['np_w'], r2(p['np_b']),
                             b0['pn_w'], r2(b0['pn_b']))
    ea0 = _tc_edge_encode(edge_attr, p['ep_w'], r2(p['ep_b']))

    g1 = _sc_gather1(t1, row)
    msg1 = _tc_message(ea0, g1, b0['pe_w1'], r2(b0['pe_b1']),
                       b0['pe_w2'].reshape(1, 32), b0['pe_b2'].reshape(1, 1))
    p1 = _sc_scatter_add(msg1, col, zeros_tile)
    s1, st1 = _tc_sum_stats(x0, p1)
    xn1, t2 = _tc_bn_relu_node(s1, st1, r2(b0['bn_ng']), r2(b0['bn_nb']),
                               b1['pn_w'], r2(b1['pn_b']))

    src, tgt, g2 = _sc_gather3(xn1, t2, row, col)
    ee_w1 = b0['ee_w1']
    y, ste = _tc_edge_mlp(
        ea0, src, tgt,
        ee_w1[0:H], ee_w1[H:2 * H], ee_w1[2 * H:3 * H], ee_w1[3 * H:4 * H],
        ee_w1[4 * H:4 * H + 1], r2(b0['ee_b1']), b0['ee_w2'], r2(b0['ee_b2']))

    msg2 = _tc_message2(y, ste, r2(b0['bn_eg']), r2(b0['bn_eb']),
                        b1['pe_w1'], r2(b1['pe_b1']),
                        b1['pe_w2'].reshape(1, 32), b1['pe_b2'].reshape(1, 1), g2)
    p2 = _sc_scatter_add(msg2, col, zeros_tile)
    s2, st2 = _tc_sum_stats(xn1, p2)
    xn2 = _tc_bn_relu_node(s2, st2, r2(b1['bn_ng']), r2(b1['bn_nb']))
    return xn2


# speedup vs baseline: 3.2419x; 1.5605x over previous
import functools

import jax
import jax.numpy as jnp
from jax import lax
from jax.experimental import pallas as pl
from jax.experimental.pallas import tpu as pltpu
from jax.experimental.pallas import tpu_sc as plsc

N = 10000
E = 320000
H = 128
NC = 2
NS = 16
NW = NC * NS
PER_W = E // NW
C = 80
CH = PER_W // C
NPAD = 10240
ROWS_PER_TILE = NPAD // NS

BN_EPS = 1e-5

def _mk_mesh():
    return plsc.VectorSubcoreMesh(core_axis_name="c", subcore_axis_name="s",
                                  num_cores=NC, num_subcores=NS)


def _sc_msg_scatter(t, om, row, col, zeros_tile):
    @functools.partial(
        pl.kernel,
        out_type=jax.ShapeDtypeStruct((NC, NPAD, H), jnp.float32),
        mesh=_mk_mesh(),
        scratch_types=[
            pltpu.VMEM((C,), jnp.int32), pltpu.VMEM((C,), jnp.int32),
            pltpu.VMEM((C,), jnp.int32), pltpu.VMEM((C,), jnp.int32),
            pltpu.VMEM((C,), jnp.float32), pltpu.VMEM((C,), jnp.float32),
            pltpu.VMEM((C, H), jnp.float32), pltpu.VMEM((C, H), jnp.float32),
            pltpu.VMEM_SHARED((NPAD, H), jnp.float32),
            pltpu.SemaphoreType.DMA, pltpu.SemaphoreType.DMA,
            pltpu.SemaphoreType.DMA, pltpu.SemaphoreType.DMA,
        ],
    )
    def k(t_hbm, om_hbm, row_hbm, col_hbm, zero_hbm, p_hbm,
          rowv0, rowv1, colv0, colv1, omv0, omv1, rows0, rows1, acc,
          isem0, isem1, gsem0, gsem1):
        cid = lax.axis_index("c")
        sid = lax.axis_index("s")
        wid = sid * NC + cid
        base = wid * PER_W
        bufs = ((rowv0, colv0, omv0, rows0, isem0, gsem0),
                (rowv1, colv1, omv1, rows1, isem1, gsem1))

        pltpu.sync_copy(zero_hbm,
                        acc.at[pl.ds(sid * ROWS_PER_TILE, ROWS_PER_TILE)])

        def idx_start(g, b):
            off = base + g * C
            rowv, colv, omv, _, isem, _ = bufs[b]
            pltpu.async_copy(row_hbm.at[pl.ds(off, C)], rowv, isem)
            pltpu.async_copy(col_hbm.at[pl.ds(off, C)], colv, isem)
            pltpu.async_copy(om_hbm.at[pl.ds(off, C)], omv, isem)

        def idx_wait(b):
            rowv, colv, omv, _, isem, _ = bufs[b]
            pltpu.make_async_copy(row_hbm.at[pl.ds(0, C)], rowv, isem).wait()
            pltpu.make_async_copy(col_hbm.at[pl.ds(0, C)], colv, isem).wait()
            pltpu.make_async_copy(om_hbm.at[pl.ds(0, C)], omv, isem).wait()

        def gather_start(b):
            rowv, _, _, rows, _, gsem = bufs[b]
            pltpu.async_copy(t_hbm.at[rowv], rows, gsem)

        def gather_wait(b):
            rowv, _, _, rows, _, gsem = bufs[b]
            pltpu.make_async_copy(t_hbm.at[rowv], rows, gsem).wait()

        def scale(b):
            _, _, omv, rows, _, _ = bufs[b]

            def body(e16, carry):
                om16 = omv[pl.ds(e16 * 16, 16)]
                for l in range(16):
                    om_vec = lax.gather(
                        om16, jnp.full((16, 1), l, jnp.int32),
                        lax.GatherDimensionNumbers(offset_dims=(),
                                                   collapsed_slice_dims=(0,),
                                                   start_index_map=(0,)),
                        (1,), mode=lax.GatherScatterMode.PROMISE_IN_BOUNDS)
                    e = e16 * 16 + l
                    for j in range(8):
                        sl = pl.ds(j * 16, 16)
                        rows[e, sl] = rows[e, sl] * om_vec
                return carry

            lax.fori_loop(0, C // 16, body, 0)

        idx_start(0, 0)
        plsc.subcore_barrier()
        idx_wait(0)
        gather_start(0)
        idx_start(1, 1)

        @pl.loop(0, CH, step=2)
        def _outer(g0):
            for b in range(2):
                g = g0 + b

                @pl.when(g < CH)
                def _():
                    _, colv, _, rows, _, _ = bufs[b]
                    gather_wait(b)

                    @pl.when(g + 1 < CH)
                    def _():
                        idx_wait(1 - b)
                        gather_start(1 - b)

                    scale(b)
                    pltpu.sync_copy(rows, acc.at[colv], add=True)

                    @pl.when(g + 2 < CH)
                    def _():
                        idx_start(g + 2, b)

        plsc.subcore_barrier()
        pltpu.sync_copy(
            acc.at[pl.ds(sid * ROWS_PER_TILE, ROWS_PER_TILE)],
            p_hbm.at[cid].at[pl.ds(sid * ROWS_PER_TILE, ROWS_PER_TILE)])

    return k(t, om, row, col, zeros_tile)


def _sc_gather2(xn, row, col):
    @functools.partial(
        pl.kernel,
        out_type=(
            jax.ShapeDtypeStruct((E, H), jnp.float32),
            jax.ShapeDtypeStruct((E, H), jnp.float32),
        ),
        mesh=_mk_mesh(),
        scratch_types=[
            pltpu.VMEM((C,), jnp.int32), pltpu.VMEM((C,), jnp.int32),
            pltpu.VMEM((C,), jnp.int32), pltpu.VMEM((C,), jnp.int32),
            pltpu.VMEM((C, H), jnp.float32), pltpu.VMEM((C, H), jnp.float32),
            pltpu.VMEM((C, H), jnp.float32), pltpu.VMEM((C, H), jnp.float32),
            pltpu.SemaphoreType.DMA, pltpu.SemaphoreType.DMA,
            pltpu.SemaphoreType.DMA, pltpu.SemaphoreType.DMA,
        ],
    )
    def k(xn_hbm, row_hbm, col_hbm, src_hbm, tgt_hbm,
          rowv0, rowv1, colv0, colv1, sb0, sb1, tb0, tb1,
          isem0, isem1, gsem0, gsem1):
        wid = lax.axis_index("s") * NC + lax.axis_index("c")
        base = wid * PER_W
        bufs = ((rowv0, colv0, sb0, tb0, isem0, gsem0),
                (rowv1, colv1, sb1, tb1, isem1, gsem1))

        def idx_start(g, b):
            off = base + g * C
            rowv, colv, _, _, isem, _ = bufs[b]
            pltpu.async_copy(row_hbm.at[pl.ds(off, C)], rowv, isem)
            pltpu.async_copy(col_hbm.at[pl.ds(off, C)], colv, isem)

        def idx_wait(b):
            rowv, colv, _, _, isem, _ = bufs[b]
            pltpu.make_async_copy(row_hbm.at[pl.ds(0, C)], rowv, isem).wait()
            pltpu.make_async_copy(col_hbm.at[pl.ds(0, C)], colv, isem).wait()

        def gather_start(b):
            rowv, colv, sb, tb, _, gsem = bufs[b]
            pltpu.async_copy(xn_hbm.at[rowv], sb, gsem)
            pltpu.async_copy(xn_hbm.at[colv], tb, gsem)

        def gather_wait(b):
            rowv, colv, sb, tb, _, gsem = bufs[b]
            pltpu.make_async_copy(xn_hbm.at[rowv], sb, gsem).wait()
            pltpu.make_async_copy(xn_hbm.at[colv], tb, gsem).wait()

        idx_start(0, 0)
        idx_wait(0)
        gather_start(0)
        idx_start(1, 1)

        @pl.loop(0, CH, step=2)
        def _outer(g0):
            for b in range(2):
                g = g0 + b

                @pl.when(g < CH)
                def _():
                    _, _, sb, tb, _, _ = bufs[b]
                    gather_wait(b)

                    @pl.when(g + 1 < CH)
                    def _():
                        idx_wait(1 - b)
                        gather_start(1 - b)

                    @pl.when(g + 2 < CH)
                    def _():
                        idx_start(g + 2, b)

                    off = base + g * C
                    pltpu.sync_copy(sb, src_hbm.at[pl.ds(off, C)])
                    pltpu.sync_copy(tb, tgt_hbm.at[pl.ds(off, C)])

    return k(xn, row, col)


BN_TILE = 1000
BE = 2000


def _relu(v):
    return jnp.maximum(v, 0.0)


def _dot(a, b):
    return jnp.dot(a, b, preferred_element_type=jnp.float32)


def _tc_node_encode(x, npw, npb, pnw, pnb):
    def k(x_ref, npw_ref, npb_ref, pnw_ref, pnb_ref, x0_ref, t1_ref):
        x0 = _relu(_dot(x_ref[...], npw_ref[...]) + npb_ref[...])
        x0_ref[...] = x0
        t1_ref[...] = _relu(_dot(x0, pnw_ref[...]) + pnb_ref[...])

    g = N // BN_TILE
    return pl.pallas_call(
        k,
        grid=(g,),
        in_specs=[
            pl.BlockSpec((BN_TILE, H), lambda i: (i, 0)),
            pl.BlockSpec((H, H), lambda i: (0, 0)),
            pl.BlockSpec((1, H), lambda i: (0, 0)),
            pl.BlockSpec((H, H), lambda i: (0, 0)),
            pl.BlockSpec((1, H), lambda i: (0, 0)),
        ],
        out_specs=[
            pl.BlockSpec((BN_TILE, H), lambda i: (i, 0)),
            pl.BlockSpec((BN_TILE, H), lambda i: (i, 0)),
        ],
        out_shape=[
            jax.ShapeDtypeStruct((N, H), jnp.float32),
            jax.ShapeDtypeStruct((N, H), jnp.float32),
        ],
    )(x, npw, npb, pnw, pnb)


def _tc_edge_encode(edge_attr, epw, epb):
    def k(ea_ref, w_ref, b_ref, out_ref):
        out_ref[...] = _relu(_dot(ea_ref[...], w_ref[...]) + b_ref[...])

    g = E // BE
    d_edge = edge_attr.shape[1]
    return pl.pallas_call(
        k,
        grid=(g,),
        in_specs=[
            pl.BlockSpec((BE, d_edge), lambda i: (i, 0)),
            pl.BlockSpec((d_edge, H), lambda i: (0, 0)),
            pl.BlockSpec((1, H), lambda i: (0, 0)),
        ],
        out_specs=pl.BlockSpec((BE, H), lambda i: (i, 0)),
        out_shape=jax.ShapeDtypeStruct((E, H), jnp.float32),
    )(edge_attr, epw, epb)


def _dotg_t(a, b):
    return lax.dot_general(a, b, (((0,), (1,)), ((), ())),
                           preferred_element_type=jnp.float32)


def _tc_omega1(ea0, pw1, pb1c, pw2r, pb2):
    def k(ea_ref, w1_ref, b1_ref, w2_ref, b2_ref, om_ref):
        hT = _relu(_dotg_t(w1_ref[...], ea_ref[...]) + b1_ref[...])
        om = _dot(w2_ref[...], hT) + b2_ref[...]
        om_ref[...] = om[None]

    g = E // BE
    return pl.pallas_call(
        k,
        grid=(g,),
        in_specs=[
            pl.BlockSpec((BE, H), lambda i: (i, 0)),
            pl.BlockSpec((H, 32), lambda i: (0, 0)),
            pl.BlockSpec((32, 1), lambda i: (0, 0)),
            pl.BlockSpec((1, 32), lambda i: (0, 0)),
            pl.BlockSpec((1, 1), lambda i: (0, 0)),
        ],
        out_specs=pl.BlockSpec((1, 1, BE), lambda i: (i, 0, 0)),
        out_shape=jax.ShapeDtypeStruct((g, 1, BE), jnp.float32),
    )(ea0, pw1, pb1c, pw2r, pb2)


def _tc_sum_stats(xin, partials):
    def k(x_ref, p_ref, s_ref, st_ref):
        s = x_ref[...] + p_ref[0] + p_ref[1]
        s_ref[...] = s
        upd = jnp.concatenate(
            [jnp.sum(s, 0)[None], jnp.sum(s * s, 0)[None],
             jnp.zeros((6, H), jnp.float32)], axis=0)

        @pl.when(pl.program_id(0) == 0)
        def _():
            st_ref[...] = jnp.zeros_like(st_ref)

        st_ref[...] += upd

    g = N // BN_TILE
    return pl.pallas_call(
        k,
        grid=(g,),
        in_specs=[
            pl.BlockSpec((BN_TILE, H), lambda i: (i, 0)),
            pl.BlockSpec((NC, BN_TILE, H), lambda i: (0, i, 0)),
        ],
        out_specs=[
            pl.BlockSpec((BN_TILE, H), lambda i: (i, 0)),
            pl.BlockSpec((8, H), lambda i: (0, 0)),
        ],
        out_shape=[
            jax.ShapeDtypeStruct((N, H), jnp.float32),
            jax.ShapeDtypeStruct((8, H), jnp.float32),
        ],
    )(xin, partials)


def _tc_bn_relu_node(s, stats, gamma, beta, pnw=None, pnb=None):
    with_t = pnw is not None

    def k(*refs):
        if with_t:
            s_ref, st_ref, g_ref, b_ref, w_ref, wb_ref, xn_ref, t_ref = refs
        else:
            s_ref, st_ref, g_ref, b_ref, xn_ref = refs
        st = st_ref[...]
        mu = st[0:1] * (1.0 / N)
        var = st[1:2] * (1.0 / N) - mu * mu
        xn = _relu(g_ref[...] * (s_ref[...] - mu) * lax.rsqrt(var + BN_EPS)
                   + b_ref[...])
        xn_ref[...] = xn
        if with_t:
            t_ref[...] = _relu(_dot(xn, w_ref[...]) + wb_ref[...])

    g = N // BN_TILE
    in_specs = [
        pl.BlockSpec((BN_TILE, H), lambda i: (i, 0)),
        pl.BlockSpec((8, H), lambda i: (0, 0)),
        pl.BlockSpec((1, H), lambda i: (0, 0)),
        pl.BlockSpec((1, H), lambda i: (0, 0)),
    ]
    args = [s, stats, gamma, beta]
    out_specs = [pl.BlockSpec((BN_TILE, H), lambda i: (i, 0))]
    out_shape = [jax.ShapeDtypeStruct((N, H), jnp.float32)]
    if with_t:
        in_specs += [pl.BlockSpec((H, H), lambda i: (0, 0)),
                     pl.BlockSpec((1, H), lambda i: (0, 0))]
        args += [pnw, pnb]
        out_specs.append(pl.BlockSpec((BN_TILE, H), lambda i: (i, 0)))
        out_shape.append(jax.ShapeDtypeStruct((N, H), jnp.float32))
    res = pl.pallas_call(
        k, grid=(g,), in_specs=in_specs, out_specs=out_specs,
        out_shape=out_shape,
    )(*args)
    return res if with_t else res[0]


def _tc_edge_mlp(ea0, src, tgt, w_ea, w_src, w_tgt, w_ds, w_cos, b1, w2, b2):
    def k(ea_ref, s_ref, t_ref, wea_ref, wsrc_ref, wtgt_ref, wds_ref,
          wcos_ref, b1_ref, w2_ref, b2_ref, y_ref, st_ref):
        s = s_ref[...]
        t = t_ref[...]
        d = jnp.abs(s - t)
        sn = jnp.sqrt(jnp.sum(s * s, axis=-1, keepdims=True))
        tn = jnp.sqrt(jnp.sum(t * t, axis=-1, keepdims=True))
        dot = jnp.sum(s * t, axis=-1, keepdims=True)
        cos = dot / jnp.maximum(sn * tn, 1e-8)
        h = _relu(_dot(ea_ref[...], wea_ref[...]) + _dot(s, wsrc_ref[...])
                  + _dot(t, wtgt_ref[...]) + _dot(d, wds_ref[...])
                  + cos * wcos_ref[...] + b1_ref[...])
        y = _dot(h, w2_ref[...]) + b2_ref[...]
        y_ref[...] = y
        upd = jnp.concatenate(
            [jnp.sum(y, 0)[None], jnp.sum(y * y, 0)[None],
             jnp.zeros((6, H), jnp.float32)], axis=0)

        @pl.when(pl.program_id(0) == 0)
        def _():
            st_ref[...] = jnp.zeros_like(st_ref)

        st_ref[...] += upd

    g = E // BE
    return pl.pallas_call(
        k,
        grid=(g,),
        in_specs=[
            pl.BlockSpec((BE, H), lambda i: (i, 0)),
            pl.BlockSpec((BE, H), lambda i: (i, 0)),
            pl.BlockSpec((BE, H), lambda i: (i, 0)),
            pl.BlockSpec((H, H), lambda i: (0, 0)),
            pl.BlockSpec((H, H), lambda i: (0, 0)),
            pl.BlockSpec((H, H), lambda i: (0, 0)),
            pl.BlockSpec((H, H), lambda i: (0, 0)),
            pl.BlockSpec((1, H), lambda i: (0, 0)),
            pl.BlockSpec((1, H), lambda i: (0, 0)),
            pl.BlockSpec((H, H), lambda i: (0, 0)),
            pl.BlockSpec((1, H), lambda i: (0, 0)),
        ],
        out_specs=[
            pl.BlockSpec((BE, H), lambda i: (i, 0)),
            pl.BlockSpec((8, H), lambda i: (0, 0)),
        ],
        out_shape=[
            jax.ShapeDtypeStruct((E, H), jnp.float32),
            jax.ShapeDtypeStruct((8, H), jnp.float32),
        ],
    )(ea0, src, tgt, w_ea, w_src, w_tgt, w_ds, w_cos, b1, w2, b2)


def _tc_omega2(y, stats, gamma, beta, pw1, pb1c, pw2r, pb2):
    def k(y_ref, st_ref, g_ref, b_ref, w1_ref, b1_ref, w2_ref, b2_ref,
          om_ref):
        st = st_ref[...]
        mu = st[0:1] * (1.0 / E)
        var = st[1:2] * (1.0 / E) - mu * mu
        ea1 = _relu(g_ref[...] * (y_ref[...] - mu) * lax.rsqrt(var + BN_EPS)
                    + b_ref[...])
        hT = _relu(_dotg_t(w1_ref[...], ea1) + b1_ref[...])
        om = _dot(w2_ref[...], hT) + b2_ref[...]
        om_ref[...] = om[None]

    g = E // BE
    return pl.pallas_call(
        k,
        grid=(g,),
        in_specs=[
            pl.BlockSpec((BE, H), lambda i: (i, 0)),
            pl.BlockSpec((8, H), lambda i: (0, 0)),
            pl.BlockSpec((1, H), lambda i: (0, 0)),
            pl.BlockSpec((1, H), lambda i: (0, 0)),
            pl.BlockSpec((H, 32), lambda i: (0, 0)),
            pl.BlockSpec((32, 1), lambda i: (0, 0)),
            pl.BlockSpec((1, 32), lambda i: (0, 0)),
            pl.BlockSpec((1, 1), lambda i: (0, 0)),
        ],
        out_specs=pl.BlockSpec((1, 1, BE), lambda i: (i, 0, 0)),
        out_shape=jax.ShapeDtypeStruct((g, 1, BE), jnp.float32),
    )(y, stats, gamma, beta, pw1, pb1c, pw2r, pb2)


def kernel(x, edge_index, edge_attr, params):
    row = edge_index[0]
    col = edge_index[1]
    p = params
    b0, b1 = p['blocks'][0], p['blocks'][1]

    def r2(v):
        return v.reshape(1, -1)

    zeros_tile = jnp.zeros((ROWS_PER_TILE, H), jnp.float32)

    x0, t1 = _tc_node_encode(x, p['np_w'], r2(p['np_b']),
                             b0['pn_w'], r2(b0['pn_b']))
    ea0 = _tc_edge_encode(edge_attr, p['ep_w'], r2(p['ep_b']))

    om1 = _tc_omega1(ea0, b0['pe_w1'], b0['pe_b1'].reshape(32, 1),
                     b0['pe_w2'].reshape(1, 32), b0['pe_b2'].reshape(1, 1))
    p1 = _sc_msg_scatter(t1, om1.reshape(E), row, col, zeros_tile)
    s1, st1 = _tc_sum_stats(x0, p1)
    xn1, t2 = _tc_bn_relu_node(s1, st1, r2(b0['bn_ng']), r2(b0['bn_nb']),
                               b1['pn_w'], r2(b1['pn_b']))

    src, tgt = _sc_gather2(xn1, row, col)
    ee_w1 = b0['ee_w1']
    y, ste = _tc_edge_mlp(
        ea0, src, tgt,
        ee_w1[0:H], ee_w1[H:2 * H], ee_w1[2 * H:3 * H], ee_w1[3 * H:4 * H],
        ee_w1[4 * H:4 * H + 1], r2(b0['ee_b1']), b0['ee_w2'], r2(b0['ee_b2']))

    om2 = _tc_omega2(y, ste, r2(b0['bn_eg']), r2(b0['bn_eb']),
                     b1['pe_w1'], b1['pe_b1'].reshape(32, 1),
                     b1['pe_w2'].reshape(1, 32), b1['pe_b2'].reshape(1, 1))
    p2 = _sc_msg_scatter(t2, om2.reshape(E), row, col, zeros_tile)
    s2, st2 = _tc_sum_stats(xn1, p2)
    xn2 = _tc_bn_relu_node(s2, st2, r2(b1['bn_ng']), r2(b1['bn_nb']))
    return xn2
